# baseline (speedup 1.0000x reference)
import functools

import jax
import jax.numpy as jnp
from jax import lax
from jax.experimental import pallas as pl
from jax.experimental.pallas import tpu as pltpu
from jax.experimental.pallas import tpu_sc as plsc

B, N, K = 4, 1024, 20
KP = 24
BN = B * N
HN = BN // 2
HB = B // 2
NEDGE = BN * K
EPS = 1e-5
BLK = 256
TP = 128
NW = 32
PW = HN // NW
PB = 128
NBLK = HN // PB


def _knn_body(b0, xt_ref, pts_ref, out_ref):
    b = pl.program_id(0)
    xt = xt_ref[0]
    pi = pts_ref[0]
    x0, x1, x2 = xt[0:1, :], xt[1:2, :], xt[2:3, :]
    p0, p1, p2 = pi[:, 0:1], pi[:, 1:2], pi[:, 2:3]
    g = jnp.dot(pi.astype(jnp.bfloat16), xt.astype(jnp.bfloat16),
                preferred_element_type=jnp.float32)
    sqj = (x0 * x0 + x1 * x1) + x2 * x2
    sqi = (p0 * p0 + p1 * p1) + p2 * p2
    d = (sqi + sqj) - 2.0 * g
    iota = lax.broadcasted_iota(jnp.int32, (BLK, N), 1)
    cols = []
    for _ in range(K):
        m = jnp.min(d, axis=1, keepdims=True)
        cand = jnp.where(d <= m, iota, N)
        amin = jnp.min(cand, axis=1, keepdims=True)
        cols.append(amin)
        d = jnp.where(iota == amin, jnp.float32(jnp.inf), d)
    cols += [cols[0]] * (KP - K)
    out_ref[...] = jnp.concatenate(cols, axis=1) + b * N


def _knn(x, b0):
    pts = jnp.transpose(x, (0, 2, 1))
    return pl.pallas_call(
        functools.partial(_knn_body, b0),
        grid=(HB, N // BLK),
        in_specs=[
            pl.BlockSpec((1, 3, N), lambda b, i: (b0 + b, 0, 0)),
            pl.BlockSpec((1, BLK, 3), lambda b, i: (b0 + b, i, 0)),
        ],
        out_specs=pl.BlockSpec((BLK, KP), lambda b, i: (b * (N // BLK) + i, 0)),
        out_shape=jax.ShapeDtypeStruct((HN, KP), jnp.int32),
    )(x, pts)


CH = 4
NCH = PW // CH
CR = CH * KP


def _make_gather():
    mesh = plsc.VectorSubcoreMesh(core_axis_name="c", subcore_axis_name="s")

    @functools.partial(
        pl.kernel,
        mesh=mesh,
        out_type=jax.ShapeDtypeStruct((HN * KP, TP), jnp.float32),
        scratch_types=[
            pltpu.VMEM((PW * KP,), jnp.int32),
            pltpu.VMEM((CR, TP), jnp.float32),
            pltpu.VMEM((CR, TP), jnp.float32),
            pltpu.VMEM((CR, TP), jnp.float32),
            pltpu.VMEM((CR, TP), jnp.float32),
            pltpu.SemaphoreType.DMA,
            pltpu.SemaphoreType.DMA,
            pltpu.SemaphoreType.DMA,
            pltpu.SemaphoreType.DMA,
            pltpu.SemaphoreType.DMA,
            pltpu.SemaphoreType.DMA,
            pltpu.SemaphoreType.DMA,
            pltpu.SemaphoreType.DMA,
        ],
    )
    def gather(x_hbm, idx_hbm, g_hbm, idx_v, r0, r1, r2, r3,
               sg0, sg1, sg2, sg3, sw0, sw1, sw2, sw3):
        wid = lax.axis_index("s") * 2 + lax.axis_index("c")
        base = wid * PW
        pltpu.sync_copy(idx_hbm.at[pl.ds(base * KP, PW * KP)], idx_v)

        rows = [r0, r1, r2, r3]
        sg = [sg0, sg1, sg2, sg3]
        sw = [sw0, sw1, sw2, sw3]

        def isl(c):
            return idx_v.at[pl.ds(c * CR, CR)]

        def gsl(c):
            return g_hbm.at[pl.ds(base * KP + c * CR, CR)]

        for j in range(NCH):
            b = j % 4
            if j >= 4:
                pltpu.make_async_copy(rows[b], gsl(j - 4), sw[b]).wait()
            pltpu.async_copy(x_hbm.at[isl(j)], rows[b], sg[b])
            if j >= 3:
                c = j - 3
                bc = c % 4
                pltpu.make_async_copy(x_hbm.at[isl(c)], rows[bc], sg[bc]).wait()
                pltpu.async_copy(rows[bc], gsl(c), sw[bc])
        for c in range(NCH - 3, NCH):
            bc = c % 4
            pltpu.make_async_copy(x_hbm.at[isl(c)], rows[bc], sg[bc]).wait()
            pltpu.async_copy(rows[bc], gsl(c), sw[bc])
        for c in range(NCH - 4, NCH):
            bc = c % 4
            pltpu.make_async_copy(rows[bc], gsl(c), sw[bc]).wait()

    return gather


def _edge_acc(diff_rows, X, Wd, Zc):
    h = jnp.dot(diff_rows(0, X).astype(jnp.bfloat16), Wd,
                preferred_element_type=jnp.float32) + Zc
    M = h
    s1 = h
    s2 = h * h
    for k in range(1, K):
        h = jnp.dot(diff_rows(k, X).astype(jnp.bfloat16), Wd,
                    preferred_element_type=jnp.float32) + Zc
        M = jnp.maximum(M, h)
        s1 = s1 + h
        s2 = s2 + h * h
    return M, s1, s2


def _edge0_body(g_ref, x_ref, zc_ref, wd_ref, mt_ref, p1_ref, p2_ref):
    X = x_ref[...]
    Zc = zc_ref[...]
    Wd = wd_ref[...].astype(jnp.bfloat16)
    M, s1, s2 = _edge_acc(lambda k, Xc: g_ref[:, k, :] - Xc, X, Wd, Zc)
    mt_ref[...] = jnp.concatenate(
        [M, jnp.zeros((PB, TP - M.shape[1]), jnp.float32)], axis=1)
    p1_ref[...] = jnp.sum(s1, axis=0, keepdims=True)[None]
    p2_ref[...] = jnp.sum(s2, axis=0, keepdims=True)[None]


def _edge_n_body(cout, g_ref, x_ref, zc_ref, wd_ref, s_ref, t_ref,
                 m_ref, p1_ref, p2_ref):
    s = s_ref[...]
    t = t_ref[...]
    Xo = jnp.maximum(x_ref[...] * s + t, 0.0)
    Zc = zc_ref[...]
    Wd = wd_ref[...].astype(jnp.bfloat16)

    def diff(k, Xc):
        return jnp.maximum(g_ref[:, k, :] * s + t, 0.0) - Xc

    M, s1, s2 = _edge_acc(diff, Xo, Wd, Zc)
    if cout < TP:
        M = jnp.concatenate(
            [M, jnp.zeros((PB, TP - cout), jnp.float32)], axis=1)
    m_ref[...] = M
    p1_ref[...] = jnp.sum(s1, axis=0, keepdims=True)[None]
    p2_ref[...] = jnp.sum(s2, axis=0, keepdims=True)[None]


def _edge0(G3, tableH, Zc, WdT, half):
    off = half * NBLK
    return pl.pallas_call(
        _edge0_body,
        grid=(NBLK,),
        in_specs=[
            pl.BlockSpec((PB, KP, TP), lambda i: (i, 0, 0)),
            pl.BlockSpec((PB, TP), lambda i: (i, 0)),
            pl.BlockSpec((PB, 64), lambda i: (off + i, 0)),
            pl.BlockSpec((TP, 64), lambda i: (0, 0)),
        ],
        out_specs=[
            pl.BlockSpec((PB, TP), lambda i: (i, 0)),
            pl.BlockSpec((1, 1, 64), lambda i: (i, 0, 0)),
            pl.BlockSpec((1, 1, 64), lambda i: (i, 0, 0)),
        ],
        out_shape=[
            jax.ShapeDtypeStruct((HN, TP), jnp.float32),
            jax.ShapeDtypeStruct((NBLK, 1, 64), jnp.float32),
            jax.ShapeDtypeStruct((NBLK, 1, 64), jnp.float32),
        ],
    )(G3, tableH, Zc, WdT)


def _edge_n(G3, tableH, Zc, WdT, sc, sh, cout, half):
    off = half * NBLK
    mw = max(cout, TP)
    return pl.pallas_call(
        functools.partial(_edge_n_body, cout),
        grid=(NBLK,),
        in_specs=[
            pl.BlockSpec((PB, KP, TP), lambda i: (i, 0, 0)),
            pl.BlockSpec((PB, TP), lambda i: (i, 0)),
            pl.BlockSpec((PB, cout), lambda i: (off + i, 0)),
            pl.BlockSpec((TP, cout), lambda i: (0, 0)),
            pl.BlockSpec((1, TP), lambda i: (0, 0)),
            pl.BlockSpec((1, TP), lambda i: (0, 0)),
        ],
        out_specs=[
            pl.BlockSpec((PB, mw), lambda i: (i, 0)),
            pl.BlockSpec((1, 1, cout), lambda i: (i, 0, 0)),
            pl.BlockSpec((1, 1, cout), lambda i: (i, 0, 0)),
        ],
        out_shape=[
            jax.ShapeDtypeStruct((HN, mw), jnp.float32),
            jax.ShapeDtypeStruct((NBLK, 1, cout), jnp.float32),
            jax.ShapeDtypeStruct((NBLK, 1, cout), jnp.float32),
        ],
    )(G3, tableH, Zc, WdT, sc, sh)


def _bn_stats(p1a, p1b, p2a, p2b, g, b):
    s1 = (jnp.sum(p1a.reshape(NBLK, -1), axis=0, keepdims=True)
          + jnp.sum(p1b.reshape(NBLK, -1), axis=0, keepdims=True))
    s2 = (jnp.sum(p2a.reshape(NBLK, -1), axis=0, keepdims=True)
          + jnp.sum(p2b.reshape(NBLK, -1), axis=0, keepdims=True))
    mean = s1 * (1.0 / NEDGE)
    var = s2 * (1.0 / NEDGE) - mean * mean
    scale = g / jnp.sqrt(var + EPS)
    shift = b - mean * scale
    return scale, shift


def _bn_mid_body(cm, ma_ref, mb_ref, p1a_ref, p1b_ref, p2a_ref, p2b_ref,
                 g_ref, b_ref, wc_ref, o_ref, zc_ref, s_ref, t_ref):
    scale, shift = _bn_stats(p1a_ref[...], p1b_ref[...],
                             p2a_ref[...], p2b_ref[...],
                             g_ref[...], b_ref[...])
    M = jnp.concatenate([ma_ref[...], mb_ref[...]], axis=0)[:, :cm]
    o = jnp.maximum(M * scale + shift, 0.0)
    o_ref[...] = o
    zc_ref[...] = jnp.dot(o.astype(jnp.bfloat16),
                          wc_ref[...].astype(jnp.bfloat16),
                          preferred_element_type=jnp.float32)
    if cm < TP:
        pad = jnp.zeros((1, TP - cm), jnp.float32)
        s_ref[...] = jnp.concatenate([scale, pad], axis=1)
        t_ref[...] = jnp.concatenate([shift, pad], axis=1)
    else:
        s_ref[...] = scale
        t_ref[...] = shift


def _bn_last_body(ma_ref, mb_ref, p1a_ref, p1b_ref, p2a_ref, p2b_ref,
                  g_ref, b_ref, o_ref):
    scale, shift = _bn_stats(p1a_ref[...], p1b_ref[...],
                             p2a_ref[...], p2b_ref[...],
                             g_ref[...], b_ref[...])
    M = jnp.concatenate([ma_ref[...], mb_ref[...]], axis=0)
    o_ref[...] = jnp.maximum(M * scale + shift, 0.0)


def _bn_mid(MA, MB, P1A, P1B, P2A, P2B, g, b, WcT, Cn2):
    C = P1A.shape[2]
    return pl.pallas_call(
        functools.partial(_bn_mid_body, C),
        out_shape=[jax.ShapeDtypeStruct((BN, C), jnp.float32),
                   jax.ShapeDtypeStruct((BN, Cn2), jnp.float32),
                   jax.ShapeDtypeStruct((1, TP), jnp.float32),
                   jax.ShapeDtypeStruct((1, TP), jnp.float32)],
    )(MA, MB, P1A, P1B, P2A, P2B, g.reshape(1, C), b.reshape(1, C), WcT)


def _bn_last(MA, MB, P1A, P1B, P2A, P2B, g, b):
    C = MA.shape[1]
    return pl.pallas_call(
        _bn_last_body,
        out_shape=jax.ShapeDtypeStruct((BN, C), jnp.float32),
    )(MA, MB, P1A, P1B, P2A, P2B, g.reshape(1, C), b.reshape(1, C))


def _prep_body(x_ref, w_ref, zc_ref):
    zc_ref[...] = jnp.dot(x_ref[...].astype(jnp.bfloat16),
                          w_ref[...].astype(jnp.bfloat16),
                          preferred_element_type=jnp.float32)


def _prep(X0p, Wc0T):
    return pl.pallas_call(
        _prep_body,
        out_shape=jax.ShapeDtypeStruct((BN, 64), jnp.float32),
    )(X0p, Wc0T)


def _final_body(o0_ref, o1_ref, o2_ref, w0_ref, w1_ref, w2_ref, g_ref, b_ref,
                out_ref):
    h = (jnp.dot(o0_ref[...].astype(jnp.bfloat16),
                 w0_ref[...].astype(jnp.bfloat16),
                 preferred_element_type=jnp.float32)
         + jnp.dot(o1_ref[...].astype(jnp.bfloat16),
                   w1_ref[...].astype(jnp.bfloat16),
                   preferred_element_type=jnp.float32)
         + jnp.dot(o2_ref[...].astype(jnp.bfloat16),
                   w2_ref[...].astype(jnp.bfloat16),
                   preferred_element_type=jnp.float32))
    mean = jnp.mean(h, axis=0, keepdims=True)
    var = jnp.mean((h - mean) * (h - mean), axis=0, keepdims=True)
    scale = g_ref[...] / jnp.sqrt(var + EPS)
    shift = b_ref[...] - mean * scale
    out_ref[...] = jnp.maximum(h * scale + shift, 0.0)


def _final(o0, o1, o2, WfT, gf, bf):
    return pl.pallas_call(
        _final_body,
        out_shape=jax.ShapeDtypeStruct((BN, 512), jnp.float32),
    )(o0, o1, o2, WfT[:64], WfT[64:192], WfT[192:448],
      gf.reshape(1, 512), bf.reshape(1, 512))


def _pad_rows(W, rows):
    return jnp.pad(W, ((0, rows - W.shape[0]), (0, 0)))


def kernel(x, W0, g0, b0, W1, g1, b1, W2, g2, b2, Wf, gf, bf):
    idxA = _knn(x, 0).reshape(HN * KP)
    idxB = _knn(x, HB).reshape(HN * KP)

    Xr = jnp.transpose(x, (0, 2, 1)).reshape(BN, 3)
    X0p = jnp.pad(Xr, ((0, 0), (0, TP - 3)))
    X0pA, X0pB = X0p[:HN], X0p[HN:]

    gth = _make_gather()

    Wd0T = _pad_rows(W0[:, :3].T, TP)
    Wc0T = _pad_rows(W0[:, 3:].T, TP)
    Zc0 = _prep(X0p, Wc0T)
    G0A = gth(X0pA, idxA).reshape(HN, KP, TP)
    G0B = gth(X0pB, idxB).reshape(HN, KP, TP)
    TA0, P1A0, P2A0 = _edge0(G0A, X0pA, Zc0, Wd0T, 0)
    TB0, P1B0, P2B0 = _edge0(G0B, X0pB, Zc0, Wd0T, 1)
    o0, Zc1, s0, t0 = _bn_mid(TA0, TB0, P1A0, P1B0, P2A0, P2B0, g0, b0,
                              W1[:, 64:].T, 128)

    Wd1T = _pad_rows(W1[:, :64].T, TP)
    G1A = gth(TA0, idxA).reshape(HN, KP, TP)
    G1B = gth(TB0, idxB).reshape(HN, KP, TP)
    TA1, P1A1, P2A1 = _edge_n(G1A, TA0, Zc1, Wd1T, s0, t0, 128, 0)
    TB1, P1B1, P2B1 = _edge_n(G1B, TB0, Zc1, Wd1T, s0, t0, 128, 1)
    o1, Zc2, s1, t1 = _bn_mid(TA1, TB1, P1A1, P1B1, P2A1, P2B1, g1, b1,
                              W2[:, 128:].T, 256)

    Wd2T = W2[:, :128].T
    G2A = gth(TA1, idxA).reshape(HN, KP, TP)
    G2B = gth(TB1, idxB).reshape(HN, KP, TP)
    MA2, P1A2, P2A2 = _edge_n(G2A, TA1, Zc2, Wd2T, s1, t1, 256, 0)
    MB2, P1B2, P2B2 = _edge_n(G2B, TB1, Zc2, Wd2T, s1, t1, 256, 1)
    o2 = _bn_last(MA2, MB2, P1A2, P1B2, P2A2, P2B2, g2, b2)

    out = _final(o0, o1, o2, Wf.T, gf, bf)
    return out.reshape(B, N, 512)


# speedup vs baseline: 1.4432x; 1.4432x over previous
import functools

import jax
import jax.numpy as jnp
from jax import lax
from jax.experimental import pallas as pl
from jax.experimental.pallas import tpu as pltpu
from jax.experimental.pallas import tpu_sc as plsc

B, N, K = 4, 1024, 20
KP = 24
BN = B * N
HN = BN // 2
HB = B // 2
NEDGE = BN * K
EPS = 1e-5
BLK = 256
TP = 128
NW = 32
PW = HN // NW
PB = 128
NBLK = HN // PB


def _knn_body(b0, xt_ref, pts_ref, out_ref):
    b = pl.program_id(0)
    xt = xt_ref[0]
    pi = pts_ref[0]
    x0, x1, x2 = xt[0:1, :], xt[1:2, :], xt[2:3, :]
    p0, p1, p2 = pi[:, 0:1], pi[:, 1:2], pi[:, 2:3]
    g = jnp.dot(pi.astype(jnp.bfloat16), xt.astype(jnp.bfloat16),
                preferred_element_type=jnp.float32)
    sqj = (x0 * x0 + x1 * x1) + x2 * x2
    sqi = (p0 * p0 + p1 * p1) + p2 * p2
    d = (sqi + sqj) - 2.0 * g
    iota = lax.broadcasted_iota(jnp.int32, (BLK, N), 1)
    cols = []
    for _ in range(K):
        m = jnp.min(d, axis=1, keepdims=True)
        cand = jnp.where(d <= m, iota, N)
        amin = jnp.min(cand, axis=1, keepdims=True)
        cols.append(amin)
        d = jnp.where(iota == amin, jnp.float32(jnp.inf), d)
    cols += [cols[0]] * (KP - K)
    out_ref[...] = jnp.concatenate(cols, axis=1) + b * N


def _knn(x, b0):
    pts = jnp.transpose(x, (0, 2, 1))
    return pl.pallas_call(
        functools.partial(_knn_body, b0),
        grid=(HB, N // BLK),
        in_specs=[
            pl.BlockSpec((1, 3, N), lambda b, i: (b0 + b, 0, 0)),
            pl.BlockSpec((1, BLK, 3), lambda b, i: (b0 + b, i, 0)),
        ],
        out_specs=pl.BlockSpec((BLK, KP), lambda b, i: (b * (N // BLK) + i, 0)),
        out_shape=jax.ShapeDtypeStruct((HN, KP), jnp.int32),
    )(x, pts)


def _make_gather():
    mesh = plsc.VectorSubcoreMesh(core_axis_name="c", subcore_axis_name="s")

    @functools.partial(
        pl.kernel,
        mesh=mesh,
        out_type=jax.ShapeDtypeStruct((K * HN, TP), jnp.float32),
        scratch_types=[
            pltpu.VMEM((K * PW,), jnp.int32),
            pltpu.VMEM((PW, TP), jnp.float32),
            pltpu.VMEM((PW, TP), jnp.float32),
            pltpu.VMEM((PW, TP), jnp.float32),
            pltpu.VMEM((PW, TP), jnp.float32),
            pltpu.SemaphoreType.DMA,
            pltpu.SemaphoreType.DMA,
            pltpu.SemaphoreType.DMA,
            pltpu.SemaphoreType.DMA,
            pltpu.SemaphoreType.DMA,
            pltpu.SemaphoreType.DMA,
            pltpu.SemaphoreType.DMA,
            pltpu.SemaphoreType.DMA,
        ],
    )
    def gather(x_hbm, idx_hbm, g_hbm, idx_v, r0, r1, r2, r3,
               sg0, sg1, sg2, sg3, sw0, sw1, sw2, sw3):
        wid = lax.axis_index("s") * 2 + lax.axis_index("c")
        base = wid * PW
        pltpu.sync_copy(idx_hbm.at[pl.ds(wid * K * PW, K * PW)], idx_v)

        rows = [r0, r1, r2, r3]
        sg = [sg0, sg1, sg2, sg3]
        sw = [sw0, sw1, sw2, sw3]

        def isl(c):
            return idx_v.at[pl.ds(c * PW, PW)]

        def gsl(c):
            return g_hbm.at[pl.ds(c * HN + base, PW)]

        for j in range(K):
            b = j % 4
            if j >= 4:
                pltpu.make_async_copy(rows[b], gsl(j - 4), sw[b]).wait()
            pltpu.async_copy(x_hbm.at[isl(j)], rows[b], sg[b])
            if j >= 3:
                c = j - 3
                bc = c % 4
                pltpu.make_async_copy(x_hbm.at[isl(c)], rows[bc], sg[bc]).wait()
                pltpu.async_copy(rows[bc], gsl(c), sw[bc])
        for c in range(K - 3, K):
            bc = c % 4
            pltpu.make_async_copy(x_hbm.at[isl(c)], rows[bc], sg[bc]).wait()
            pltpu.async_copy(rows[bc], gsl(c), sw[bc])
        for c in range(K - 4, K):
            bc = c % 4
            pltpu.make_async_copy(rows[bc], gsl(c), sw[bc]).wait()

    return gather


def _edge_acc(diff_rows, X, Wd, Zc):
    h = jnp.dot(diff_rows(0, X).astype(jnp.bfloat16), Wd,
                preferred_element_type=jnp.float32) + Zc
    M = h
    s1 = h
    s2 = h * h
    for k in range(1, K):
        h = jnp.dot(diff_rows(k, X).astype(jnp.bfloat16), Wd,
                    preferred_element_type=jnp.float32) + Zc
        M = jnp.maximum(M, h)
        s1 = s1 + h
        s2 = s2 + h * h
    return M, s1, s2


def _edge0_body(g_ref, x_ref, zc_ref, wd_ref, mt_ref, p1_ref, p2_ref):
    X = x_ref[...]
    Zc = zc_ref[...]
    Wd = wd_ref[...].astype(jnp.bfloat16)
    M, s1, s2 = _edge_acc(lambda k, Xc: g_ref[k] - Xc, X, Wd, Zc)
    mt_ref[...] = jnp.concatenate(
        [M, jnp.zeros((PB, TP - M.shape[1]), jnp.float32)], axis=1)
    p1_ref[...] = jnp.sum(s1, axis=0, keepdims=True)[None]
    p2_ref[...] = jnp.sum(s2, axis=0, keepdims=True)[None]


def _edge_n_body(cout, g_ref, x_ref, zc_ref, wd_ref, s_ref, t_ref,
                 m_ref, p1_ref, p2_ref):
    s = s_ref[...]
    t = t_ref[...]
    Xo = jnp.maximum(x_ref[...] * s + t, 0.0)
    Zc = zc_ref[...]
    Wd = wd_ref[...].astype(jnp.bfloat16)

    def diff(k, Xc):
        return jnp.maximum(g_ref[k] * s + t, 0.0) - Xc

    M, s1, s2 = _edge_acc(diff, Xo, Wd, Zc)
    if cout < TP:
        M = jnp.concatenate(
            [M, jnp.zeros((PB, TP - cout), jnp.float32)], axis=1)
    m_ref[...] = M
    p1_ref[...] = jnp.sum(s1, axis=0, keepdims=True)[None]
    p2_ref[...] = jnp.sum(s2, axis=0, keepdims=True)[None]


def _edge0(G3, tableH, Zc, WdT, half):
    off = half * NBLK
    return pl.pallas_call(
        _edge0_body,
        grid=(NBLK,),
        in_specs=[
            pl.BlockSpec((K, PB, TP), lambda i: (0, i, 0)),
            pl.BlockSpec((PB, TP), lambda i: (i, 0)),
            pl.BlockSpec((PB, 64), lambda i: (off + i, 0)),
            pl.BlockSpec((TP, 64), lambda i: (0, 0)),
        ],
        out_specs=[
            pl.BlockSpec((PB, TP), lambda i: (i, 0)),
            pl.BlockSpec((1, 1, 64), lambda i: (i, 0, 0)),
            pl.BlockSpec((1, 1, 64), lambda i: (i, 0, 0)),
        ],
        out_shape=[
            jax.ShapeDtypeStruct((HN, TP), jnp.float32),
            jax.ShapeDtypeStruct((NBLK, 1, 64), jnp.float32),
            jax.ShapeDtypeStruct((NBLK, 1, 64), jnp.float32),
        ],
    )(G3, tableH, Zc, WdT)


def _edge_n(G3, tableH, Zc, WdT, sc, sh, cout, half):
    off = half * NBLK
    mw = max(cout, TP)
    return pl.pallas_call(
        functools.partial(_edge_n_body, cout),
        grid=(NBLK,),
        in_specs=[
            pl.BlockSpec((K, PB, TP), lambda i: (0, i, 0)),
            pl.BlockSpec((PB, TP), lambda i: (i, 0)),
            pl.BlockSpec((PB, cout), lambda i: (off + i, 0)),
            pl.BlockSpec((TP, cout), lambda i: (0, 0)),
            pl.BlockSpec((1, TP), lambda i: (0, 0)),
            pl.BlockSpec((1, TP), lambda i: (0, 0)),
        ],
        out_specs=[
            pl.BlockSpec((PB, mw), lambda i: (i, 0)),
            pl.BlockSpec((1, 1, cout), lambda i: (i, 0, 0)),
            pl.BlockSpec((1, 1, cout), lambda i: (i, 0, 0)),
        ],
        out_shape=[
            jax.ShapeDtypeStruct((HN, mw), jnp.float32),
            jax.ShapeDtypeStruct((NBLK, 1, cout), jnp.float32),
            jax.ShapeDtypeStruct((NBLK, 1, cout), jnp.float32),
        ],
    )(G3, tableH, Zc, WdT, sc, sh)


def _bn_stats(p1a, p1b, p2a, p2b, g, b):
    s1 = (jnp.sum(p1a.reshape(NBLK, -1), axis=0, keepdims=True)
          + jnp.sum(p1b.reshape(NBLK, -1), axis=0, keepdims=True))
    s2 = (jnp.sum(p2a.reshape(NBLK, -1), axis=0, keepdims=True)
          + jnp.sum(p2b.reshape(NBLK, -1), axis=0, keepdims=True))
    mean = s1 * (1.0 / NEDGE)
    var = s2 * (1.0 / NEDGE) - mean * mean
    scale = g / jnp.sqrt(var + EPS)
    shift = b - mean * scale
    return scale, shift


def _bn_mid_body(cm, ma_ref, mb_ref, p1a_ref, p1b_ref, p2a_ref, p2b_ref,
                 g_ref, b_ref, wc_ref, o_ref, zc_ref, s_ref, t_ref):
    scale, shift = _bn_stats(p1a_ref[...], p1b_ref[...],
                             p2a_ref[...], p2b_ref[...],
                             g_ref[...], b_ref[...])
    M = jnp.concatenate([ma_ref[...], mb_ref[...]], axis=0)[:, :cm]
    o = jnp.maximum(M * scale + shift, 0.0)
    o_ref[...] = o
    zc_ref[...] = jnp.dot(o.astype(jnp.bfloat16),
                          wc_ref[...].astype(jnp.bfloat16),
                          preferred_element_type=jnp.float32)
    if cm < TP:
        pad = jnp.zeros((1, TP - cm), jnp.float32)
        s_ref[...] = jnp.concatenate([scale, pad], axis=1)
        t_ref[...] = jnp.concatenate([shift, pad], axis=1)
    else:
        s_ref[...] = scale
        t_ref[...] = shift


def _bn_last_body(ma_ref, mb_ref, p1a_ref, p1b_ref, p2a_ref, p2b_ref,
                  g_ref, b_ref, o_ref):
    scale, shift = _bn_stats(p1a_ref[...], p1b_ref[...],
                             p2a_ref[...], p2b_ref[...],
                             g_ref[...], b_ref[...])
    M = jnp.concatenate([ma_ref[...], mb_ref[...]], axis=0)
    o_ref[...] = jnp.maximum(M * scale + shift, 0.0)


def _bn_mid(MA, MB, P1A, P1B, P2A, P2B, g, b, WcT, Cn2):
    C = P1A.shape[2]
    return pl.pallas_call(
        functools.partial(_bn_mid_body, C),
        out_shape=[jax.ShapeDtypeStruct((BN, C), jnp.float32),
                   jax.ShapeDtypeStruct((BN, Cn2), jnp.float32),
                   jax.ShapeDtypeStruct((1, TP), jnp.float32),
                   jax.ShapeDtypeStruct((1, TP), jnp.float32)],
    )(MA, MB, P1A, P1B, P2A, P2B, g.reshape(1, C), b.reshape(1, C), WcT)


def _bn_last(MA, MB, P1A, P1B, P2A, P2B, g, b):
    C = MA.shape[1]
    return pl.pallas_call(
        _bn_last_body,
        out_shape=jax.ShapeDtypeStruct((BN, C), jnp.float32),
    )(MA, MB, P1A, P1B, P2A, P2B, g.reshape(1, C), b.reshape(1, C))


def _prep_body(x_ref, w_ref, zc_ref):
    zc_ref[...] = jnp.dot(x_ref[...].astype(jnp.bfloat16),
                          w_ref[...].astype(jnp.bfloat16),
                          preferred_element_type=jnp.float32)


def _prep(X0p, Wc0T):
    return pl.pallas_call(
        _prep_body,
        out_shape=jax.ShapeDtypeStruct((BN, 64), jnp.float32),
    )(X0p, Wc0T)


def _final_body(o0_ref, o1_ref, o2_ref, w0_ref, w1_ref, w2_ref, g_ref, b_ref,
                out_ref):
    h = (jnp.dot(o0_ref[...].astype(jnp.bfloat16),
                 w0_ref[...].astype(jnp.bfloat16),
                 preferred_element_type=jnp.float32)
         + jnp.dot(o1_ref[...].astype(jnp.bfloat16),
                   w1_ref[...].astype(jnp.bfloat16),
                   preferred_element_type=jnp.float32)
         + jnp.dot(o2_ref[...].astype(jnp.bfloat16),
                   w2_ref[...].astype(jnp.bfloat16),
                   preferred_element_type=jnp.float32))
    mean = jnp.mean(h, axis=0, keepdims=True)
    var = jnp.mean((h - mean) * (h - mean), axis=0, keepdims=True)
    scale = g_ref[...] / jnp.sqrt(var + EPS)
    shift = b_ref[...] - mean * scale
    out_ref[...] = jnp.maximum(h * scale + shift, 0.0)


def _final(o0, o1, o2, WfT, gf, bf):
    return pl.pallas_call(
        _final_body,
        out_shape=jax.ShapeDtypeStruct((BN, 512), jnp.float32),
    )(o0, o1, o2, WfT[:64], WfT[64:192], WfT[192:448],
      gf.reshape(1, 512), bf.reshape(1, 512))


def _pad_rows(W, rows):
    return jnp.pad(W, ((0, rows - W.shape[0]), (0, 0)))


def _perm_idx(idx):
    return (idx[:, :K].reshape(NW, PW, K).transpose(0, 2, 1)
            .reshape(NW * K * PW))


def kernel(x, W0, g0, b0, W1, g1, b1, W2, g2, b2, Wf, gf, bf):
    idxA = _perm_idx(_knn(x, 0))
    idxB = _perm_idx(_knn(x, HB))

    Xr = jnp.transpose(x, (0, 2, 1)).reshape(BN, 3)
    X0p = jnp.pad(Xr, ((0, 0), (0, TP - 3)))
    X0pA, X0pB = X0p[:HN], X0p[HN:]

    gth = _make_gather()

    Wd0T = _pad_rows(W0[:, :3].T, TP)
    Wc0T = _pad_rows(W0[:, 3:].T, TP)
    Zc0 = _prep(X0p, Wc0T)
    G0A = gth(X0pA, idxA).reshape(K, HN, TP)
    G0B = gth(X0pB, idxB).reshape(K, HN, TP)
    TA0, P1A0, P2A0 = _edge0(G0A, X0pA, Zc0, Wd0T, 0)
    TB0, P1B0, P2B0 = _edge0(G0B, X0pB, Zc0, Wd0T, 1)
    o0, Zc1, s0, t0 = _bn_mid(TA0, TB0, P1A0, P1B0, P2A0, P2B0, g0, b0,
                              W1[:, 64:].T, 128)

    Wd1T = _pad_rows(W1[:, :64].T, TP)
    G1A = gth(TA0, idxA).reshape(K, HN, TP)
    G1B = gth(TB0, idxB).reshape(K, HN, TP)
    TA1, P1A1, P2A1 = _edge_n(G1A, TA0, Zc1, Wd1T, s0, t0, 128, 0)
    TB1, P1B1, P2B1 = _edge_n(G1B, TB0, Zc1, Wd1T, s0, t0, 128, 1)
    o1, Zc2, s1, t1 = _bn_mid(TA1, TB1, P1A1, P1B1, P2A1, P2B1, g1, b1,
                              W2[:, 128:].T, 256)

    Wd2T = W2[:, :128].T
    G2A = gth(TA1, idxA).reshape(K, HN, TP)
    G2B = gth(TB1, idxB).reshape(K, HN, TP)
    MA2, P1A2, P2A2 = _edge_n(G2A, TA1, Zc2, Wd2T, s1, t1, 256, 0)
    MB2, P1B2, P2B2 = _edge_n(G2B, TB1, Zc2, Wd2T, s1, t1, 256, 1)
    o2 = _bn_last(MA2, MB2, P1A2, P1B2, P2A2, P2B2, g2, b2)

    out = _final(o0, o1, o2, Wf.T, gf, bf)
    return out.reshape(B, N, 512)
